# baseline (speedup 1.0000x reference)
import functools

import jax
import jax.numpy as jnp
from jax.experimental import pallas as pl
from jax.experimental.pallas import tpu as pltpu

NUM_TOKENS = 32768
MODEL_DIM = 768
NUM_EXPERTS = 8
TILE_T = 8192


def _router_kernel(x_ref, w_ref, comb_ref, laux_ref, acc_ref, *, num_tiles, num_tokens):
    i = pl.program_id(0)
    e = NUM_EXPERTS

    x = x_ref[...]
    w = w_ref[...]
    logits = jax.lax.dot_general(
        x, w, (((1,), (1,)), ((), ())), preferred_element_type=jnp.float32
    )
    lt = logits.T

    m = jnp.max(lt, axis=0, keepdims=True)
    ex = jnp.exp(lt - m)
    gates = ex / jnp.sum(ex, axis=0, keepdims=True)

    gmax = jnp.max(gates, axis=0, keepdims=True)
    is_max = (gates == gmax)
    iota = jax.lax.broadcasted_iota(jnp.int32, gates.shape, 0)
    idx = jnp.min(jnp.where(is_max, iota, e), axis=0, keepdims=True)
    onehot = (iota == idx).astype(jnp.float32)

    comb_ref[...] = gates * onehot

    @pl.when(i == 0)
    def _init():
        acc_ref[...] = jnp.zeros_like(acc_ref)

    acc_ref[:, 0:1] += jnp.sum(gates, axis=1, keepdims=True)
    acc_ref[:, 1:2] += jnp.sum(onehot, axis=1, keepdims=True)

    @pl.when(i == num_tiles - 1)
    def _finish():
        me = acc_ref[:, 0:1] / num_tokens
        ce = acc_ref[:, 1:2] / num_tokens
        laux_ref[...] = jnp.sum(me * ce).reshape(1, 1) * e


def kernel(x, W):
    t, d = x.shape
    e = W.shape[0]
    num_tiles = t // TILE_T
    comb, laux = pl.pallas_call(
        functools.partial(_router_kernel, num_tiles=num_tiles, num_tokens=t),
        grid=(num_tiles,),
        in_specs=[
            pl.BlockSpec((TILE_T, d), lambda i: (i, 0)),
            pl.BlockSpec((e, d), lambda i: (0, 0)),
        ],
        out_specs=[
            pl.BlockSpec((e, TILE_T), lambda i: (0, i)),
            pl.BlockSpec((1, 1), lambda i: (0, 0)),
        ],
        out_shape=[
            jax.ShapeDtypeStruct((e, t), jnp.float32),
            jax.ShapeDtypeStruct((1, 1), jnp.float32),
        ],
        scratch_shapes=[pltpu.VMEM((e, 2), jnp.float32)],
    )(x, W)
    return laux[0, 0], comb


# speedup vs baseline: 1.0785x; 1.0785x over previous
import functools

import jax
import jax.numpy as jnp
from jax.experimental import pallas as pl
from jax.experimental.pallas import tpu as pltpu

NUM_TOKENS = 32768
MODEL_DIM = 768
NUM_EXPERTS = 8
TILE_T = 4096


def _router_kernel(x_ref, w_ref, comb_ref, laux_ref, acc_ref, *, num_tiles, num_tokens):
    i = pl.program_id(0)
    e = NUM_EXPERTS

    x = x_ref[...]
    w = w_ref[...]
    logits = jax.lax.dot_general(
        x, w, (((1,), (1,)), ((), ())), preferred_element_type=jnp.float32
    )
    lt = logits.T

    m = jnp.max(lt, axis=0, keepdims=True)
    ex = jnp.exp(lt - m)
    gates = ex / jnp.sum(ex, axis=0, keepdims=True)

    gmax = jnp.max(gates, axis=0, keepdims=True)
    is_max = (gates == gmax)
    iota = jax.lax.broadcasted_iota(jnp.int32, gates.shape, 0)
    idx = jnp.min(jnp.where(is_max, iota, e), axis=0, keepdims=True)
    onehot = (iota == idx).astype(jnp.float32)

    comb_ref[...] = gates * onehot

    @pl.when(i == 0)
    def _init():
        acc_ref[...] = jnp.zeros_like(acc_ref)

    acc_ref[:, 0:1] += jnp.sum(gates, axis=1, keepdims=True)
    acc_ref[:, 1:2] += jnp.sum(onehot, axis=1, keepdims=True)

    @pl.when(i == num_tiles - 1)
    def _finish():
        me = acc_ref[:, 0:1] / num_tokens
        ce = acc_ref[:, 1:2] / num_tokens
        laux_ref[...] = jnp.sum(me * ce).reshape(1, 1) * e


def kernel(x, W):
    t, d = x.shape
    e = W.shape[0]
    num_tiles = t // TILE_T
    comb, laux = pl.pallas_call(
        functools.partial(_router_kernel, num_tiles=num_tiles, num_tokens=t),
        grid=(num_tiles,),
        in_specs=[
            pl.BlockSpec((TILE_T, d), lambda i: (i, 0)),
            pl.BlockSpec((e, d), lambda i: (0, 0)),
        ],
        out_specs=[
            pl.BlockSpec((e, TILE_T), lambda i: (0, i)),
            pl.BlockSpec((1, 1), lambda i: (0, 0)),
        ],
        out_shape=[
            jax.ShapeDtypeStruct((e, t), jnp.float32),
            jax.ShapeDtypeStruct((1, 1), jnp.float32),
        ],
        scratch_shapes=[pltpu.VMEM((e, 2), jnp.float32)],
    )(x, W)
    return laux[0, 0], comb
